# baseline (speedup 1.0000x reference)
import functools
from typing import Any

import jax
import jax.numpy as jnp
from jax import lax
from jax.experimental import pallas as pl
from jax.experimental.pallas import tpu as pltpu

T = 256
D = 1024
NH = 16
NKV = 4
HD = 64
E = 8
TOPK = 2
I = 768
EPS = 1e-06
THETA = 1000000.0

BLK = 64
NBLK = 16
NPAD = NBLK * BLK

_NEG = -3.0e38


def _attn_kernel(h_ref, cos_ref, sin_ref, wqkv_ref, qn_ref, kn_ref, wo_ref,
                 winln_ref, wpostln_ref, wgate_ref,
                 h2_ref, xn2_ref, e1_ref, e2_ref, w1_ref, w2_ref):
    h = h_ref[...]
    var = jnp.mean(h * h, axis=-1, keepdims=True)
    xn = h * lax.rsqrt(var + EPS) * winln_ref[...]
    qkv = jnp.dot(xn, wqkv_ref[...], preferred_element_type=jnp.float32)

    cos = cos_ref[...]
    sin = sin_ref[...]

    def norm_rope(x, w):
        v = jnp.mean(x * x, axis=-1, keepdims=True)
        x = x * lax.rsqrt(v + EPS) * w
        x1 = x[:, : HD // 2]
        x2 = x[:, HD // 2:]
        return jnp.concatenate([x1 * cos - x2 * sin, x2 * cos + x1 * sin],
                               axis=1)

    row = lax.broadcasted_iota(jnp.int32, (T, T), 0)
    col = lax.broadcasted_iota(jnp.int32, (T, T), 1)
    causal = col <= row

    kv_base = NH * HD
    ks = []
    vs = []
    for j in range(NKV):
        kj = qkv[:, kv_base + j * HD: kv_base + (j + 1) * HD]
        ks.append(norm_rope(kj, kn_ref[...]))
        vs.append(qkv[:, kv_base + NKV * HD + j * HD:
                      kv_base + NKV * HD + (j + 1) * HD])

    heads = []
    scale = HD ** -0.5
    for hd_i in range(NH):
        q = norm_rope(qkv[:, hd_i * HD: (hd_i + 1) * HD], qn_ref[...])
        k = ks[hd_i // (NH // NKV)]
        v = vs[hd_i // (NH // NKV)]
        s = lax.dot_general(q, k, (((1,), (1,)), ((), ())),
                            preferred_element_type=jnp.float32) * scale
        s = jnp.where(causal, s, _NEG)
        m = jnp.max(s, axis=-1, keepdims=True)
        p = jnp.exp(s - m)
        p = p / jnp.sum(p, axis=-1, keepdims=True)
        heads.append(jnp.dot(p, v, preferred_element_type=jnp.float32))

    attn = jnp.concatenate(heads, axis=1)
    h2 = h + jnp.dot(attn, wo_ref[...], preferred_element_type=jnp.float32)
    h2_ref[...] = h2

    var2 = jnp.mean(h2 * h2, axis=-1, keepdims=True)
    xn2 = h2 * lax.rsqrt(var2 + EPS) * wpostln_ref[...]
    xn2_ref[...] = xn2

    logits = jnp.dot(xn2, wgate_ref[...], preferred_element_type=jnp.float32)
    ids = lax.broadcasted_iota(jnp.int32, (T, E), 1)
    m1 = jnp.max(logits, axis=-1, keepdims=True)
    i1 = jnp.min(jnp.where(logits == m1, ids, E + 1), axis=-1, keepdims=True)
    l2 = jnp.where(ids == i1, _NEG, logits)
    m2 = jnp.max(l2, axis=-1, keepdims=True)
    i2 = jnp.min(jnp.where((logits == m2) & (ids != i1), ids, E + 1),
                 axis=-1, keepdims=True)
    r = jnp.exp(m2 - m1)
    w1 = 1.0 / (1.0 + r)
    e1_ref[...] = i1
    e2_ref[...] = i2
    w1_ref[...] = w1
    w2_ref[...] = 1.0 - w1


def _run_attn(h, cosT, sinT, w_qkv, qn, kn, w_o, w_in_ln, w_post_ln, w_gate):
    out_shapes = (
        jax.ShapeDtypeStruct((T, D), jnp.float32),
        jax.ShapeDtypeStruct((T, D), jnp.float32),
        jax.ShapeDtypeStruct((T, 1), jnp.int32),
        jax.ShapeDtypeStruct((T, 1), jnp.int32),
        jax.ShapeDtypeStruct((T, 1), jnp.float32),
        jax.ShapeDtypeStruct((T, 1), jnp.float32),
    )
    return pl.pallas_call(
        _attn_kernel,
        out_shape=out_shapes,
    )(h, cosT, sinT, w_qkv, qn, kn, w_o, w_in_ln, w_post_ln, w_gate)


def _ffn_kernel(be_ref, bv_ref, xg_ref, wgu_ref, wd_ref, wpad_ref, out_ref):
    b = pl.program_id(0)

    @pl.when(bv_ref[b] == 1)
    def _():
        x = xg_ref[...]
        gu = jnp.dot(x, wgu_ref[0], preferred_element_type=jnp.float32)
        g = gu[:, :I]
        u = gu[:, I:]
        act = g * (1.0 / (1.0 + jnp.exp(-g))) * u
        dout = jnp.dot(act, wd_ref[0], preferred_element_type=jnp.float32)
        out_ref[...] = dout * wpad_ref[...]


def _run_ffn(block_expert, block_valid, xg, w_gate_up, w_down, w_pad):
    grid_spec = pltpu.PrefetchScalarGridSpec(
        num_scalar_prefetch=2,
        grid=(NBLK,),
        in_specs=[
            pl.BlockSpec((BLK, D), lambda b, be, bv: (b, 0)),
            pl.BlockSpec((1, D, 2 * I), lambda b, be, bv: (be[b], 0, 0)),
            pl.BlockSpec((1, I, D), lambda b, be, bv: (be[b], 0, 0)),
            pl.BlockSpec((BLK, 1), lambda b, be, bv: (b, 0)),
        ],
        out_specs=pl.BlockSpec((BLK, D), lambda b, be, bv: (b, 0)),
    )
    return pl.pallas_call(
        _ffn_kernel,
        grid_spec=grid_spec,
        out_shape=jax.ShapeDtypeStruct((NPAD, D), jnp.float32),
    )(block_expert, block_valid, xg, w_gate_up, w_down, w_pad)


def _route_meta(e1, e2, w1, w2):
    ef = jnp.concatenate([e1, e2])
    wf = jnp.concatenate([w1, w2])
    tok = jnp.concatenate([jnp.arange(T, dtype=jnp.int32)] * 2)
    onehot = (ef[:, None] == jnp.arange(E)[None, :]).astype(jnp.int32)
    counts = jnp.sum(onehot, axis=0)
    ranks_mat = jnp.cumsum(onehot, axis=0) - onehot
    rank = jnp.sum(ranks_mat * onehot, axis=1)
    nb = (counts + BLK - 1) // BLK
    pad_off = BLK * (jnp.cumsum(nb) - nb)
    dest = pad_off[ef] + rank
    tok_pad = jnp.zeros((NPAD,), jnp.int32).at[dest].set(tok)
    w_pad = jnp.zeros((NPAD,), jnp.float32).at[dest].set(wf)
    nblk_tot = jnp.sum(nb)
    bvec = jnp.arange(NBLK, dtype=jnp.int32)
    blk_of = pad_off // BLK
    be = jnp.zeros((NBLK,), jnp.int32)
    for e in range(E):
        be = jnp.where((bvec >= blk_of[e]) & (bvec < blk_of[e] + nb[e]),
                       e, be)
    last_e = jnp.argmax(jnp.where(counts > 0, jnp.arange(E), -1))
    be = jnp.where(bvec >= nblk_tot, last_e, be).astype(jnp.int32)
    bv = (bvec < nblk_tot).astype(jnp.int32)
    inv1 = dest[:T]
    inv2 = dest[T:]
    return tok_pad, w_pad, be, bv, inv1, inv2


def kernel(positions, hidden_states, w_in_ln, w_qkv, q_norm_w, k_norm_w,
           w_o, w_post_ln, w_gate, w_gate_up, w_down):
    pos = positions.astype(jnp.float32)
    inv_freq = 1.0 / (THETA ** (jnp.arange(0, HD, 2, dtype=jnp.float32) / HD))
    freqs = pos[:, None] * inv_freq[None, :]
    cosT = jnp.cos(freqs)
    sinT = jnp.sin(freqs)

    h2, xn2, e1, e2, w1, w2 = _run_attn(
        hidden_states, cosT, sinT, w_qkv,
        q_norm_w.reshape(1, HD), k_norm_w.reshape(1, HD), w_o,
        w_in_ln.reshape(1, D), w_post_ln.reshape(1, D), w_gate)

    tok_pad, w_pad, be, bv, inv1, inv2 = _route_meta(
        e1.reshape(T), e2.reshape(T), w1.reshape(T), w2.reshape(T))

    xg = jnp.take(xn2, tok_pad, axis=0)
    dout = _run_ffn(be, bv, xg, w_gate_up, w_down, w_pad.reshape(NPAD, 1))
    return h2 + dout[inv1] + dout[inv2]
